# baseline (speedup 1.0000x reference)
import functools

import jax
import jax.numpy as jnp
from jax import lax
from jax.experimental import pallas as pl
from jax.experimental.pallas import tpu as pltpu
from jax.experimental.pallas import tpu_sc as plsc

N = 10000
E = 320000
INC = 128
OUTC = 128
K = 27

NC = 2
NS = 16
NW = NC * NS

EPW = E // NW
CHUNK = 80
NCHUNK = EPW // CHUNK
NP = 10240
RPT = NP // NS


def _matmul_body(f_ref, w_ref, o_ref):
    o_ref[...] = jnp.dot(f_ref[...], w_ref[0],
                         preferred_element_type=jnp.float32)


def _fx_all_offsets(features, kernel):
    return pl.pallas_call(
        _matmul_body,
        grid=(K,),
        in_specs=[
            pl.BlockSpec((N, INC), lambda k: (0, 0)),
            pl.BlockSpec((1, INC, OUTC), lambda k: (k, 0, 0)),
        ],
        out_specs=pl.BlockSpec((N, OUTC), lambda k: (k, 0)),
        out_shape=jax.ShapeDtypeStruct((K * N, OUTC), jnp.float32),
    )(features, kernel)


_sc_mesh = plsc.VectorSubcoreMesh(core_axis_name="c", subcore_axis_name="s")


NBUF = 3
EPG = NBUF * CHUNK
NG = NCHUNK // NBUF
NTAIL = NCHUNK - NG * NBUF


@functools.partial(
    pl.kernel,
    out_type=jax.ShapeDtypeStruct((NC, NP, OUTC), jnp.float32),
    mesh=_sc_mesh,
    scratch_types=(
        [pltpu.VMEM((EPG,), jnp.int32) for _ in range(2)]
        + [pltpu.VMEM((CHUNK,), jnp.int32) for _ in range(2 * NBUF)]
        + [pltpu.VMEM((CHUNK, OUTC), jnp.float32) for _ in range(NBUF)]
        + [pltpu.VMEM_SHARED((NP, OUTC), jnp.float32)]
        + [pltpu.SemaphoreType.DMA for _ in range(3 + 2 * NBUF)]
    ),
)
def _sc_gather_scatter(fx_hbm, gidx_hbm, didx_hbm, zro_hbm, out_hbm, *rest):
    idxg = rest[:2]
    idxd = [rest[2:2 + NBUF], rest[2 + NBUF:2 + 2 * NBUF]]
    rows = rest[2 + 2 * NBUF:2 + 3 * NBUF]
    acc = rest[2 + 3 * NBUF]
    isem = rest[3 + 3 * NBUF:5 + 3 * NBUF]
    gsem = rest[5 + 3 * NBUF:5 + 4 * NBUF]
    ssem = rest[5 + 4 * NBUF:5 + 5 * NBUF]
    zsem = rest[5 + 5 * NBUF]

    cid = lax.axis_index("c")
    sid = lax.axis_index("s")
    wid = cid * NS + sid
    base = wid * EPW

    pltpu.async_copy(zro_hbm, acc.at[pl.ds(sid * RPT, RPT)], zsem)

    def fire_idx(g, bank):
        off = base + g * EPG
        pltpu.async_copy(gidx_hbm.at[pl.ds(off, EPG)], idxg[bank], isem[bank])
        for b in range(NBUF):
            pltpu.async_copy(didx_hbm.at[pl.ds(off + b * CHUNK, CHUNK)],
                             idxd[bank][b], isem[bank])

    def wait_idx(bank):
        pltpu.make_async_copy(gidx_hbm.at[pl.ds(0, EPG)], idxg[bank],
                              isem[bank]).wait()
        for b in range(NBUF):
            pltpu.make_async_copy(didx_hbm.at[pl.ds(0, CHUNK)],
                                  idxd[bank][b], isem[bank]).wait()

    def fire_gather(bank, b):
        pltpu.async_copy(fx_hbm.at[idxg[bank].at[pl.ds(b * CHUNK, CHUNK)]],
                         rows[b], gsem[b])

    def wait_gather(bank, b):
        pltpu.make_async_copy(fx_hbm.at[idxg[bank].at[pl.ds(b * CHUNK, CHUNK)]],
                              rows[b], gsem[b]).wait()

    def fire_scatter(bank, b):
        pltpu.async_copy(rows[b], acc.at[idxd[bank][b]], ssem[b], add=True)

    def wait_scatter(bank, b):
        pltpu.make_async_copy(rows[b], acc.at[idxd[bank][b]], ssem[b]).wait()

    fire_idx(0, 0)
    fire_idx(1, 1)
    pltpu.make_async_copy(zro_hbm, acc.at[pl.ds(sid * RPT, RPT)], zsem).wait()
    plsc.subcore_barrier()
    wait_idx(0)
    for b in range(NBUF):
        fire_gather(0, b)

    def pair_body(g2, carry):
        for gg in range(2):
            g = 2 * g2 + gg
            nxt = 1 - gg
            for b in range(NBUF):
                wait_gather(gg, b)
                fire_scatter(gg, b)
            wait_idx(nxt)
            for b in range(NBUF):
                wait_scatter(gg, b)
                fire_gather(nxt, b)
            fire_idx(g + 2, gg)
        return carry

    lax.fori_loop(0, (NG - 3) // 2, pair_body, 0)

    for b in range(NBUF):
        wait_gather(0, b)
        fire_scatter(0, b)
    wait_idx(1)
    for b in range(NBUF):
        wait_scatter(0, b)
        fire_gather(1, b)
    fire_idx(NG - 1, 0)
    for b in range(NBUF):
        wait_gather(1, b)
        fire_scatter(1, b)
    wait_idx(0)
    for b in range(NBUF):
        wait_scatter(1, b)
        fire_gather(0, b)
    for b in range(NBUF):
        wait_gather(0, b)
        fire_scatter(0, b)
    for b in range(NBUF):
        wait_scatter(0, b)

    for t in range(NTAIL):
        j = NG * NBUF + t
        pltpu.sync_copy(gidx_hbm.at[pl.ds(base + j * CHUNK, CHUNK)],
                        idxd[1][t])
        pltpu.sync_copy(didx_hbm.at[pl.ds(base + j * CHUNK, CHUNK)],
                        idxd[0][t])
        pltpu.async_copy(fx_hbm.at[idxd[1][t]], rows[t], gsem[t]).wait()
        pltpu.async_copy(rows[t], acc.at[idxd[0][t]], ssem[t],
                         add=True).wait()

    plsc.subcore_barrier()
    pltpu.sync_copy(acc.at[pl.ds(sid * RPT, RPT)],
                    out_hbm.at[cid, pl.ds(sid * RPT, RPT)])


def _sum_body(p_ref, o_ref):
    o_ref[...] = p_ref[0] + p_ref[1]


def _sum_partials(partials):
    bn = 2000
    return pl.pallas_call(
        _sum_body,
        grid=(N // bn,),
        in_specs=[pl.BlockSpec((NC, bn, OUTC), lambda i: (0, i, 0))],
        out_specs=pl.BlockSpec((bn, OUTC), lambda i: (i, 0)),
        out_shape=jax.ShapeDtypeStruct((N, OUTC), jnp.float32),
    )(partials)


@jax.jit
def kernel(features, coords, edge_index, kernel_offsets, kernel):
    del coords
    fx2 = _fx_all_offsets(features, kernel)

    src = edge_index[0]
    dst = edge_index[1]
    gidx = kernel_offsets * N + src
    didx = dst
    zro = jnp.zeros((RPT, OUTC), jnp.float32)

    partials = _sc_gather_scatter(fx2, gidx, didx, zro)
    return _sum_partials(partials)


# speedup vs baseline: 1.1276x; 1.1276x over previous
import functools

import jax
import jax.numpy as jnp
from jax import lax
from jax.experimental import pallas as pl
from jax.experimental.pallas import tpu as pltpu
from jax.experimental.pallas import tpu_sc as plsc

N = 10000
E = 320000
INC = 128
OUTC = 128
K = 27

NC = 2
NS = 16
NW = NC * NS

EPW = E // NW
CHUNK = 40
NCHUNK = EPW // CHUNK
NP = 10240
RPT = NP // NS


def _matmul_body(f_ref, w_ref, ei_ref, ko_ref, o_ref, gi_ref, di_ref):
    o_ref[...] = jnp.dot(f_ref[...], w_ref[0],
                         preferred_element_type=jnp.float32)

    @pl.when(pl.program_id(0) == 0)
    def _():
        gi_ref[...] = ko_ref[...] * N + ei_ref[0]
        di_ref[...] = ei_ref[1]


def _fx_all_offsets(features, kernel, edge_index, kernel_offsets):
    return pl.pallas_call(
        _matmul_body,
        grid=(K,),
        in_specs=[
            pl.BlockSpec((N, INC), lambda k: (0, 0)),
            pl.BlockSpec((1, INC, OUTC), lambda k: (k, 0, 0)),
            pl.BlockSpec((2, E), lambda k: (0, 0)),
            pl.BlockSpec((E,), lambda k: (0,)),
        ],
        out_specs=[
            pl.BlockSpec((N, OUTC), lambda k: (k, 0)),
            pl.BlockSpec((E,), lambda k: (0,)),
            pl.BlockSpec((E,), lambda k: (0,)),
        ],
        out_shape=[
            jax.ShapeDtypeStruct((K * N, OUTC), jnp.float32),
            jax.ShapeDtypeStruct((E,), jnp.int32),
            jax.ShapeDtypeStruct((E,), jnp.int32),
        ],
    )(features, kernel, edge_index, kernel_offsets)


_sc_mesh = plsc.VectorSubcoreMesh(core_axis_name="c", subcore_axis_name="s")


NBUF = 5
EPG = NBUF * CHUNK
NG = NCHUNK // NBUF


@functools.partial(
    pl.kernel,
    out_type=jax.ShapeDtypeStruct((NC, NP, OUTC), jnp.float32),
    mesh=_sc_mesh,
    scratch_types=(
        [pltpu.VMEM((EPG,), jnp.int32) for _ in range(2)]
        + [pltpu.VMEM((CHUNK,), jnp.int32) for _ in range(2 * NBUF)]
        + [pltpu.VMEM((CHUNK, OUTC), jnp.float32) for _ in range(NBUF)]
        + [pltpu.VMEM_SHARED((NP, OUTC), jnp.float32)]
        + [pltpu.SemaphoreType.DMA for _ in range(3 + 2 * NBUF)]
    ),
)
def _sc_gather_scatter(fx_hbm, gidx_hbm, didx_hbm, zro_hbm, out_hbm, *rest):
    idxg = rest[:2]
    idxd = [rest[2:2 + NBUF], rest[2 + NBUF:2 + 2 * NBUF]]
    rows = rest[2 + 2 * NBUF:2 + 3 * NBUF]
    acc = rest[2 + 3 * NBUF]
    isem = rest[3 + 3 * NBUF:5 + 3 * NBUF]
    gsem = rest[5 + 3 * NBUF:5 + 4 * NBUF]
    ssem = rest[5 + 4 * NBUF:5 + 5 * NBUF]
    zsem = rest[5 + 5 * NBUF]

    cid = lax.axis_index("c")
    sid = lax.axis_index("s")
    wid = cid * NS + sid
    base = wid * EPW

    pltpu.async_copy(zro_hbm, acc.at[pl.ds(sid * RPT, RPT)], zsem)

    def fire_idx(g, bank):
        off = base + g * EPG
        pltpu.async_copy(gidx_hbm.at[pl.ds(off, EPG)], idxg[bank], isem[bank])
        for b in range(NBUF):
            pltpu.async_copy(didx_hbm.at[pl.ds(off + b * CHUNK, CHUNK)],
                             idxd[bank][b], isem[bank])

    def wait_idx(bank):
        pltpu.make_async_copy(gidx_hbm.at[pl.ds(0, EPG)], idxg[bank],
                              isem[bank]).wait()
        for b in range(NBUF):
            pltpu.make_async_copy(didx_hbm.at[pl.ds(0, CHUNK)],
                                  idxd[bank][b], isem[bank]).wait()

    def fire_gather(bank, b):
        pltpu.async_copy(fx_hbm.at[idxg[bank].at[pl.ds(b * CHUNK, CHUNK)]],
                         rows[b], gsem[b])

    def wait_gather(bank, b):
        pltpu.make_async_copy(fx_hbm.at[idxg[bank].at[pl.ds(b * CHUNK, CHUNK)]],
                              rows[b], gsem[b]).wait()

    def fire_scatter(bank, b):
        pltpu.async_copy(rows[b], acc.at[idxd[bank][b]], ssem[b], add=True)

    def wait_scatter(bank, b):
        pltpu.make_async_copy(rows[b], acc.at[idxd[bank][b]], ssem[b]).wait()

    fire_idx(0, 0)
    fire_idx(1, 1)
    pltpu.make_async_copy(zro_hbm, acc.at[pl.ds(sid * RPT, RPT)], zsem).wait()
    plsc.subcore_barrier()
    wait_idx(0)
    for b in range(NBUF):
        fire_gather(0, b)

    def pair_body(g2, carry):
        for gg in range(2):
            g = 2 * g2 + gg
            nxt = 1 - gg
            for b in range(NBUF):
                wait_gather(gg, b)
                fire_scatter(gg, b)
            wait_idx(nxt)
            for b in range(NBUF):
                wait_scatter(gg, b)
                fire_gather(nxt, b)
            fire_idx(g + 2, gg)
        return carry

    lax.fori_loop(0, NG // 2 - 1, pair_body, 0)

    for b in range(NBUF):
        wait_gather(0, b)
        fire_scatter(0, b)
    wait_idx(1)
    for b in range(NBUF):
        wait_scatter(0, b)
        fire_gather(1, b)
    for b in range(NBUF):
        wait_gather(1, b)
        fire_scatter(1, b)
    for b in range(NBUF):
        wait_scatter(1, b)

    plsc.subcore_barrier()
    pltpu.sync_copy(acc.at[pl.ds(sid * RPT, RPT)],
                    out_hbm.at[cid, pl.ds(sid * RPT, RPT)])


def _sum_body(p_ref, o_ref):
    o_ref[...] = p_ref[0] + p_ref[1]


def _sum_partials(partials):
    bn = 2000
    return pl.pallas_call(
        _sum_body,
        grid=(N // bn,),
        in_specs=[pl.BlockSpec((NC, bn, OUTC), lambda i: (0, i, 0))],
        out_specs=pl.BlockSpec((bn, OUTC), lambda i: (i, 0)),
        out_shape=jax.ShapeDtypeStruct((N, OUTC), jnp.float32),
    )(partials)


@jax.jit
def kernel(features, coords, edge_index, kernel_offsets, kernel):
    del coords
    fx2, gidx, didx = _fx_all_offsets(features, kernel, edge_index,
                                      kernel_offsets)
    zro = jnp.zeros((RPT, OUTC), jnp.float32)

    partials = _sc_gather_scatter(fx2, gidx, didx, zro)
    return _sum_partials(partials)
